# Initial kernel scaffold; baseline (speedup 1.0000x reference)
import jax
import jax.numpy as jnp
from jax.experimental import pallas as pl


def kernel(x, edge_index, W1, b1, W2, b2, Wo, bo):
    raise NotImplementedError("write your pallas kernel here")


# speedup vs baseline: 5.8708x; 5.8708x over previous
import functools

import jax
import jax.numpy as jnp
from jax import lax
from jax.experimental import pallas as pl
from jax.experimental.pallas import tpu as pltpu
from jax.experimental.pallas import tpu_sc as plsc

NC = 2
NS = 16
NW = NC * NS
LANES = 128
IB = 8
KG = 4
DEG_D = 16
TRASH = 64


def _spmm_sc(N, D, R, NH, TR0, acc_rows, z_rows):
    mesh = plsc.VectorSubcoreMesh(
        core_axis_name="c", subcore_axis_name="s",
        num_cores=NC, num_subcores=NS)
    RW = R // NS

    @functools.partial(
        pl.kernel,
        out_type=jax.ShapeDtypeStruct((NC, acc_rows, D), jnp.float32),
        mesh=mesh,
        compiler_params=pltpu.CompilerParams(needs_layout_passes=False),
        scratch_types=[
            pltpu.VMEM((IB, LANES), jnp.int32),
            pltpu.VMEM((IB, LANES), jnp.int32),
            pltpu.VMEM((KG, LANES, D), jnp.float32),
            pltpu.VMEM((z_rows, D), jnp.float32),
            pltpu.VMEM_SHARED((acc_rows, D), jnp.float32),
            pltpu.SemaphoreType.DMA,
            pltpu.SemaphoreType.DMA,
        ],
    )
    def spmm(g_hbm, src_hbm, dst_hbm, out_hbm, idxs, idxd, rows, zbuf, acc,
             gsem, ssem):
        c = lax.axis_index("c")
        s = lax.axis_index("s")
        rows_per_tile = acc_rows // NS
        base = jnp.full((16,), NH, jnp.int32) * c

        def zrow(r, carry):
            for j in range(D // 16):
                zbuf[r, pl.ds(j * 16, 16)] = jnp.zeros((16,), jnp.float32)
            return carry
        lax.fori_loop(0, z_rows, zrow, 0)

        def zcopy(i, carry):
            pltpu.sync_copy(
                zbuf, acc.at[pl.ds(s * rows_per_tile + i * z_rows, z_rows)])
            return carry
        lax.fori_loop(0, rows_per_tile // z_rows, zcopy, 0)
        plsc.subcore_barrier()

        def step(i, carry):
            rb = s * RW + i * IB
            pltpu.sync_copy(src_hbm.at[pl.ds(rb, IB)], idxs)
            pltpu.sync_copy(dst_hbm.at[pl.ds(rb, IB)], idxd)
            for r in range(IB):
                for k in range(LANES // 16):
                    v = idxd[r, pl.ds(k * 16, 16)]
                    t = v - base
                    ok = (t >= 0) & (t < NH)
                    trash = (v & (TRASH - 1)) + TR0
                    idxd[r, pl.ds(k * 16, 16)] = jnp.where(ok, t, trash)
            for half in range(IB // KG):
                gd = [pltpu.async_copy(g_hbm.at[idxs.at[half * KG + j]],
                                       rows.at[j], gsem)
                      for j in range(KG)]
                for d in gd:
                    d.wait()
                sd = [pltpu.async_copy(rows.at[j],
                                       acc.at[idxd.at[half * KG + j]],
                                       ssem, add=True)
                      for j in range(KG)]
                for d in sd:
                    d.wait()
            return carry
        lax.fori_loop(0, RW // IB, step, 0)
        plsc.subcore_barrier()

        pltpu.sync_copy(acc.at[pl.ds(s * rows_per_tile, rows_per_tile)],
                        out_hbm.at[c, pl.ds(s * rows_per_tile, rows_per_tile)])

    return spmm


def _deg_sc(N, R, hp_rows):
    mesh = plsc.VectorSubcoreMesh(
        core_axis_name="c", subcore_axis_name="s",
        num_cores=NC, num_subcores=NS)
    RW = R // NW

    @functools.partial(
        pl.kernel,
        out_type=jax.ShapeDtypeStruct((NW, hp_rows, LANES), jnp.float32),
        mesh=mesh,
        compiler_params=pltpu.CompilerParams(needs_layout_passes=False),
        scratch_types=[
            pltpu.VMEM((IB, LANES), jnp.int32),
            pltpu.VMEM((hp_rows, LANES), jnp.float32),
        ],
    )
    def deg(dst_hbm, out_hbm, idxd, histo):
        c = lax.axis_index("c")
        s = lax.axis_index("s")
        wid = c * NS + s

        def zrow(r, carry):
            for k in range(LANES // 16):
                histo[r, pl.ds(k * 16, 16)] = jnp.zeros((16,), jnp.float32)
            return carry
        lax.fori_loop(0, hp_rows, zrow, 0)

        ones16 = jnp.ones((16,), jnp.float32)

        def step(i, carry):
            rb = wid * RW + i * IB
            pltpu.sync_copy(dst_hbm.at[pl.ds(rb, IB)], idxd)
            for r in range(IB):
                for k in range(LANES // 16):
                    v = idxd[r, pl.ds(k * 16, 16)]
                    rowi = lax.shift_right_logical(v, 7)
                    coli = v & (LANES - 1)
                    plsc.addupdate_scatter(histo, [rowi, coli], ones16)
            return carry
        lax.fori_loop(0, RW // IB, step, 0)

        pltpu.sync_copy(histo, out_hbm.at[wid])

    return deg


def _tc_first(N, Din, Dh, BR):
    def body(x_ref, w_ref, dp_ref, o_ref):
        deg = jnp.sum(dp_ref[...], axis=1, keepdims=True) + 1.0
        dis = lax.rsqrt(deg)
        h = jnp.dot(x_ref[...], w_ref[...], preferred_element_type=jnp.float32)
        o_ref[...] = h * dis

    return pl.pallas_call(
        body,
        grid=(N // BR,),
        in_specs=[
            pl.BlockSpec((BR, Din), lambda i: (i, 0)),
            pl.BlockSpec((Din, Dh), lambda i: (0, 0)),
            pl.BlockSpec((BR, NW), lambda i: (i, 0)),
        ],
        out_specs=pl.BlockSpec((BR, Dh), lambda i: (i, 0)),
        out_shape=jax.ShapeDtypeStruct((N, Dh), jnp.float32),
    )


def _tc_mid(N, Dh, Dn, BR, BPC):
    def body(p_ref, g_ref, dp_ref, w_ref, b_ref, o_ref):
        deg = jnp.sum(dp_ref[...], axis=1, keepdims=True) + 1.0
        dis = lax.rsqrt(deg)
        t = (p_ref[0] + g_ref[...]) * dis + b_ref[...]
        a = jnp.maximum(t, 0.0)
        h = jnp.dot(a, w_ref[...], preferred_element_type=jnp.float32)
        g2 = h * dis
        o_ref[...] = jnp.concatenate(
            [g2, jnp.zeros((g2.shape[0], Dh - Dn), jnp.float32)], axis=-1)

    return pl.pallas_call(
        body,
        grid=(N // BR,),
        in_specs=[
            pl.BlockSpec((1, BR, Dh), lambda i: (i // BPC, i % BPC, 0)),
            pl.BlockSpec((BR, Dh), lambda i: (i, 0)),
            pl.BlockSpec((BR, NW), lambda i: (i, 0)),
            pl.BlockSpec((Dh, Dn), lambda i: (0, 0)),
            pl.BlockSpec((1, Dh), lambda i: (0, 0)),
        ],
        out_specs=pl.BlockSpec((BR, Dh), lambda i: (i, 0)),
        out_shape=jax.ShapeDtypeStruct((N, Dh), jnp.float32),
    )


def _tc_last(N, Dp, Dh, Do, BR, BPC):
    def body(p_ref, g_ref, dp_ref, b_ref, wo_ref, bo_ref, o_ref):
        deg = jnp.sum(dp_ref[...], axis=1, keepdims=True) + 1.0
        dis = lax.rsqrt(deg)
        t = (p_ref[0][:, :Dh] + g_ref[...][:, :Dh]) * dis + b_ref[...]
        a = jnp.maximum(t, 0.0)
        o_ref[...] = (jnp.dot(a, wo_ref[...], preferred_element_type=jnp.float32)
                      + bo_ref[...])

    return pl.pallas_call(
        body,
        grid=(N // BR,),
        in_specs=[
            pl.BlockSpec((1, BR, Dp), lambda i: (i // BPC, i % BPC, 0)),
            pl.BlockSpec((BR, Dp), lambda i: (i, 0)),
            pl.BlockSpec((BR, NW), lambda i: (i, 0)),
            pl.BlockSpec((1, Dh), lambda i: (0, 0)),
            pl.BlockSpec((Dh, Do), lambda i: (0, 0)),
            pl.BlockSpec((1, Do), lambda i: (0, 0)),
        ],
        out_specs=pl.BlockSpec((BR, Do), lambda i: (i, 0)),
        out_shape=jax.ShapeDtypeStruct((N, Do), jnp.float32),
    )


def kernel(x, edge_index, W1, b1, W2, b2, Wo, bo):
    N, Din = x.shape
    E = edge_index.shape[1]
    Dh1 = W1.shape[1]
    Dh2 = W2.shape[1]
    Do = Wo.shape[1]

    r_raw = -(-E // LANES)
    R = -(-r_raw // (NW * IB)) * NW * IB
    e_pad = R * LANES - E
    src = edge_index[0].astype(jnp.int32)
    dst = edge_index[1].astype(jnp.int32)
    src_p = jnp.concatenate([src, jnp.zeros((e_pad,), jnp.int32)]).reshape(R, LANES)
    dst_p = jnp.concatenate([dst, jnp.full((e_pad,), N, jnp.int32)]).reshape(R, LANES)

    NH = -(-N // NC)
    TR0 = -(-NH // TRASH) * TRASH
    z_rows = 64
    acc_rows = -(-(TR0 + TRASH) // (NS * z_rows)) * NS * z_rows
    hp_rows = -(-(N + 1) // LANES)
    hp_rows += (-hp_rows) % 8

    BR = 1000 if (N // NC) % 1000 == 0 else 500
    BPC = (N // NC) // BR
    b1r = b1.reshape(1, Dh1)
    b2r = b2.reshape(1, Dh2)
    do_pad = max(8, Do)
    wo_p = jnp.zeros((Dh2, do_pad), jnp.float32).at[:, :Do].set(Wo)
    bo_p = jnp.zeros((1, do_pad), jnp.float32).at[0, :Do].set(bo)

    deg_parts = _deg_sc(N, R, hp_rows)(dst_p)
    deg_t = jnp.transpose(deg_parts.reshape(NW, hp_rows * LANES))
    g1 = _tc_first(N, Din, Dh1, BR)(x, W1, deg_t)
    p1 = _spmm_sc(N, Dh1, R, NH, TR0, acc_rows, z_rows)(g1, src_p, dst_p)
    g2 = _tc_mid(N, Dh1, Dh2, BR, BPC)(p1, g1, deg_t, W2, b1r)
    p2 = _spmm_sc(N, Dh1, R, NH, TR0, acc_rows, z_rows)(g2, src_p, dst_p)
    out = _tc_last(N, Dh1, Dh2, do_pad, BR, BPC)(p2, g2, deg_t, b2r, wo_p, bo_p)
    return out[:, :Do]


# speedup vs baseline: 6.1063x; 1.0401x over previous
import functools

import jax
import jax.numpy as jnp
from jax import lax
from jax.experimental import pallas as pl
from jax.experimental.pallas import tpu as pltpu
from jax.experimental.pallas import tpu_sc as plsc

NC = 2
NS = 16
NW = NC * NS
LANES = 128
IB = 16
KG = 4
DEG_D = 16
TRASH = 64


def _spmm_sc(N, D, R, NH, TR0, acc_rows, z_rows):
    mesh = plsc.VectorSubcoreMesh(
        core_axis_name="c", subcore_axis_name="s",
        num_cores=NC, num_subcores=NS)
    RW = R // NS

    @functools.partial(
        pl.kernel,
        out_type=jax.ShapeDtypeStruct((NC, acc_rows, D), jnp.float32),
        mesh=mesh,
        compiler_params=pltpu.CompilerParams(needs_layout_passes=False),
        scratch_types=[
            pltpu.VMEM((IB, LANES), jnp.int32),
            pltpu.VMEM((IB, LANES), jnp.int32),
            pltpu.VMEM((KG, LANES, D), jnp.float32),
            pltpu.VMEM((z_rows, D), jnp.float32),
            pltpu.VMEM_SHARED((acc_rows, D), jnp.float32),
            pltpu.SemaphoreType.DMA,
            pltpu.SemaphoreType.DMA,
            pltpu.SemaphoreType.DMA,
        ],
    )
    def spmm(g_hbm, src_hbm, dst_hbm, out_hbm, idxs, idxd, rows, zbuf, acc,
             gsem, ssem0, ssem1):
        c = lax.axis_index("c")
        s = lax.axis_index("s")
        rows_per_tile = acc_rows // NS
        base = jnp.full((16,), NH, jnp.int32) * c
        ssems = (ssem0, ssem1)

        def zrow(r, carry):
            for j in range(D // 16):
                zbuf[r, pl.ds(j * 16, 16)] = jnp.zeros((16,), jnp.float32)
            return carry
        lax.fori_loop(0, z_rows, zrow, 0)

        def zcopy(i, carry):
            pltpu.sync_copy(
                zbuf, acc.at[pl.ds(s * rows_per_tile + i * z_rows, z_rows)])
            return carry
        lax.fori_loop(0, rows_per_tile // z_rows, zcopy, 0)
        plsc.subcore_barrier()

        def step(i, carry):
            rb = s * RW + i * IB
            pltpu.sync_copy(src_hbm.at[pl.ds(rb, IB)], idxs)
            pltpu.sync_copy(dst_hbm.at[pl.ds(rb, IB)], idxd)
            for r in range(IB):
                for k in range(LANES // 16):
                    v = idxd[r, pl.ds(k * 16, 16)]
                    t = v - base
                    ok = (t >= 0) & (t < NH)
                    trash = (v & (TRASH - 1)) + TR0
                    idxd[r, pl.ds(k * 16, 16)] = jnp.where(ok, t, trash)
            pending = {}
            for grp in range(IB // 2):
                p = grp & 1
                b0, b1 = 2 * p, 2 * p + 1
                r0, r1 = 2 * grp, 2 * grp + 1
                for d in pending.pop(p, ()):
                    d.wait()
                gd0 = pltpu.async_copy(g_hbm.at[idxs.at[r0]], rows.at[b0], gsem)
                gd1 = pltpu.async_copy(g_hbm.at[idxs.at[r1]], rows.at[b1], gsem)
                gd0.wait()
                gd1.wait()
                pending[p] = (
                    pltpu.async_copy(rows.at[b0], acc.at[idxd.at[r0]],
                                     ssems[p], add=True),
                    pltpu.async_copy(rows.at[b1], acc.at[idxd.at[r1]],
                                     ssems[p], add=True),
                )
            for p in sorted(pending):
                for d in pending[p]:
                    d.wait()
            return carry
        lax.fori_loop(0, RW // IB, step, 0)
        plsc.subcore_barrier()

        pltpu.sync_copy(acc.at[pl.ds(s * rows_per_tile, rows_per_tile)],
                        out_hbm.at[c, pl.ds(s * rows_per_tile, rows_per_tile)])

    return spmm


def _deg_sc(N, R, hp_rows):
    mesh = plsc.VectorSubcoreMesh(
        core_axis_name="c", subcore_axis_name="s",
        num_cores=NC, num_subcores=NS)
    RW = R // NW

    @functools.partial(
        pl.kernel,
        out_type=jax.ShapeDtypeStruct((NW, hp_rows, LANES), jnp.float32),
        mesh=mesh,
        compiler_params=pltpu.CompilerParams(needs_layout_passes=False),
        scratch_types=[
            pltpu.VMEM((IB, LANES), jnp.int32),
            pltpu.VMEM((hp_rows, LANES), jnp.float32),
        ],
    )
    def deg(dst_hbm, out_hbm, idxd, histo):
        c = lax.axis_index("c")
        s = lax.axis_index("s")
        wid = c * NS + s

        def zrow(r, carry):
            for k in range(LANES // 16):
                histo[r, pl.ds(k * 16, 16)] = jnp.zeros((16,), jnp.float32)
            return carry
        lax.fori_loop(0, hp_rows, zrow, 0)

        ones16 = jnp.ones((16,), jnp.float32)

        def step(i, carry):
            rb = wid * RW + i * IB
            pltpu.sync_copy(dst_hbm.at[pl.ds(rb, IB)], idxd)
            for r in range(IB):
                for k in range(LANES // 16):
                    v = idxd[r, pl.ds(k * 16, 16)]
                    rowi = lax.shift_right_logical(v, 7)
                    coli = v & (LANES - 1)
                    plsc.addupdate_scatter(histo, [rowi, coli], ones16)
            return carry
        lax.fori_loop(0, RW // IB, step, 0)

        pltpu.sync_copy(histo, out_hbm.at[wid])

    return deg


def _tc_first(N, Din, Dh, BR):
    def body(x_ref, w_ref, dp_ref, o_ref):
        deg = jnp.sum(dp_ref[...], axis=1, keepdims=True) + 1.0
        dis = lax.rsqrt(deg)
        h = jnp.dot(x_ref[...], w_ref[...], preferred_element_type=jnp.float32)
        o_ref[...] = h * dis

    return pl.pallas_call(
        body,
        grid=(N // BR,),
        in_specs=[
            pl.BlockSpec((BR, Din), lambda i: (i, 0)),
            pl.BlockSpec((Din, Dh), lambda i: (0, 0)),
            pl.BlockSpec((BR, NW), lambda i: (i, 0)),
        ],
        out_specs=pl.BlockSpec((BR, Dh), lambda i: (i, 0)),
        out_shape=jax.ShapeDtypeStruct((N, Dh), jnp.float32),
    )


def _tc_mid(N, Dh, Dn, BR, BPC):
    def body(p_ref, g_ref, dp_ref, w_ref, b_ref, o_ref):
        deg = jnp.sum(dp_ref[...], axis=1, keepdims=True) + 1.0
        dis = lax.rsqrt(deg)
        t = (p_ref[0] + g_ref[...]) * dis + b_ref[...]
        a = jnp.maximum(t, 0.0)
        h = jnp.dot(a, w_ref[...], preferred_element_type=jnp.float32)
        g2 = h * dis
        o_ref[...] = jnp.concatenate(
            [g2, jnp.zeros((g2.shape[0], Dh - Dn), jnp.float32)], axis=-1)

    return pl.pallas_call(
        body,
        grid=(N // BR,),
        in_specs=[
            pl.BlockSpec((1, BR, Dh), lambda i: (i // BPC, i % BPC, 0)),
            pl.BlockSpec((BR, Dh), lambda i: (i, 0)),
            pl.BlockSpec((BR, NW), lambda i: (i, 0)),
            pl.BlockSpec((Dh, Dn), lambda i: (0, 0)),
            pl.BlockSpec((1, Dh), lambda i: (0, 0)),
        ],
        out_specs=pl.BlockSpec((BR, Dh), lambda i: (i, 0)),
        out_shape=jax.ShapeDtypeStruct((N, Dh), jnp.float32),
    )


def _tc_last(N, Dp, Dh, Do, BR, BPC):
    def body(p_ref, g_ref, dp_ref, b_ref, wo_ref, bo_ref, o_ref):
        deg = jnp.sum(dp_ref[...], axis=1, keepdims=True) + 1.0
        dis = lax.rsqrt(deg)
        t = (p_ref[0][:, :Dh] + g_ref[...][:, :Dh]) * dis + b_ref[...]
        a = jnp.maximum(t, 0.0)
        o_ref[...] = (jnp.dot(a, wo_ref[...], preferred_element_type=jnp.float32)
                      + bo_ref[...])

    return pl.pallas_call(
        body,
        grid=(N // BR,),
        in_specs=[
            pl.BlockSpec((1, BR, Dp), lambda i: (i // BPC, i % BPC, 0)),
            pl.BlockSpec((BR, Dp), lambda i: (i, 0)),
            pl.BlockSpec((BR, NW), lambda i: (i, 0)),
            pl.BlockSpec((1, Dh), lambda i: (0, 0)),
            pl.BlockSpec((Dh, Do), lambda i: (0, 0)),
            pl.BlockSpec((1, Do), lambda i: (0, 0)),
        ],
        out_specs=pl.BlockSpec((BR, Do), lambda i: (i, 0)),
        out_shape=jax.ShapeDtypeStruct((N, Do), jnp.float32),
    )


def kernel(x, edge_index, W1, b1, W2, b2, Wo, bo):
    N, Din = x.shape
    E = edge_index.shape[1]
    Dh1 = W1.shape[1]
    Dh2 = W2.shape[1]
    Do = Wo.shape[1]

    r_raw = -(-E // LANES)
    R = -(-r_raw // (NW * IB)) * NW * IB
    e_pad = R * LANES - E
    src = edge_index[0].astype(jnp.int32)
    dst = edge_index[1].astype(jnp.int32)
    src_p = jnp.concatenate([src, jnp.zeros((e_pad,), jnp.int32)]).reshape(R, LANES)
    dst_p = jnp.concatenate([dst, jnp.full((e_pad,), N, jnp.int32)]).reshape(R, LANES)

    NH = -(-N // NC)
    TR0 = -(-NH // TRASH) * TRASH
    z_rows = 64
    acc_rows = -(-(TR0 + TRASH) // (NS * z_rows)) * NS * z_rows
    hp_rows = -(-(N + 1) // LANES)
    hp_rows += (-hp_rows) % 8

    BR = 1000 if (N // NC) % 1000 == 0 else 500
    BPC = (N // NC) // BR
    b1r = b1.reshape(1, Dh1)
    b2r = b2.reshape(1, Dh2)
    do_pad = max(8, Do)
    wo_p = jnp.zeros((Dh2, do_pad), jnp.float32).at[:, :Do].set(Wo)
    bo_p = jnp.zeros((1, do_pad), jnp.float32).at[0, :Do].set(bo)

    deg_parts = _deg_sc(N, R, hp_rows)(dst_p)
    deg_t = jnp.transpose(deg_parts.reshape(NW, hp_rows * LANES))
    g1 = _tc_first(N, Din, Dh1, BR)(x, W1, deg_t)
    p1 = _spmm_sc(N, Dh1, R, NH, TR0, acc_rows, z_rows)(g1, src_p, dst_p)
    g2 = _tc_mid(N, Dh1, Dh2, BR, BPC)(p1, g1, deg_t, W2, b1r)
    p2 = _spmm_sc(N, Dh1, R, NH, TR0, acc_rows, z_rows)(g2, src_p, dst_p)
    out = _tc_last(N, Dh1, Dh2, do_pad, BR, BPC)(p2, g2, deg_t, b2r, wo_p, bo_p)
    return out[:, :Do]
